# baseline (speedup 1.0000x reference)
import jax
import jax.numpy as jnp
from jax import lax
from jax.experimental import pallas as pl
from jax.experimental.pallas import tpu as pltpu
from jax.experimental.pallas import tpu_sc as plsc

N = 10000
E = 320000
D_IN = 128
D_H = 16

NC = 2
NS = 16
NW = NC * NS
NPAD = 10240

CL = 128
CHUNKS = 80
EWP = CHUNKS * CL
EP = NW * EWP

NG = 8
GCHUNKS = EP // NG // CL
ROUNDS = 4
RCHUNKS = GCHUNKS // ROUNDS
NF = D_H // (NW // NG)

def _deg_body(src_hbm, dst_hbm, zeros_hbm, degs_hbm, src_v, dst_v, hs_v, hd_v):
    c = lax.axis_index("c")
    s = lax.axis_index("s")
    wid = c * NS + s

    pltpu.sync_copy(zeros_hbm.at[0], hs_v)
    pltpu.sync_copy(zeros_hbm.at[1], hd_v)

    pltpu.sync_copy(src_hbm.at[wid], src_v)
    pltpu.sync_copy(dst_hbm.at[wid], dst_v)

    ones = jnp.ones((16,), jnp.float32)

    def body(j, carry):
        for kk in range(CL // 16):
            sv = src_v[j, pl.ds(kk * 16, 16)]
            dv = dst_v[j, pl.ds(kk * 16, 16)]
            plsc.addupdate_scatter(hs_v, [sv], ones)
            plsc.addupdate_scatter(hd_v, [dv], ones)
        return carry

    lax.fori_loop(0, CHUNKS, body, 0)

    pltpu.sync_copy(hs_v, degs_hbm.at[0, wid])
    pltpu.sync_copy(hd_v, degs_hbm.at[1, wid])


def _degrees(src_p, dst_p, zeros):
    mesh = plsc.VectorSubcoreMesh(core_axis_name="c", subcore_axis_name="s")
    return pl.kernel(
        _deg_body,
        out_type=jax.ShapeDtypeStruct((2, NW, NPAD), jnp.float32),
        mesh=mesh,
        compiler_params=pltpu.CompilerParams(use_tc_tiling_on_sc=False, needs_layout_passes=False),
        scratch_types=[
            pltpu.VMEM((CHUNKS, CL), jnp.int32),
            pltpu.VMEM((CHUNKS, CL), jnp.int32),
            pltpu.VMEM((NPAD,), jnp.float32),
            pltpu.VMEM((NPAD,), jnp.float32),
        ],
    )(src_p, dst_p, zeros)


def _agg_body(ht_hbm, src_hbm, dst_hbm, zeros_hbm, aggp_hbm, tab_v, agg_v,
              src_v, dst_v):
    c = lax.axis_index("c")
    s = lax.axis_index("s")
    wid = c * NS + s
    g = lax.rem(wid, NG)
    fset = lax.div(wid, NG)

    pltpu.sync_copy(zeros_hbm, agg_v)

    pltpu.sync_copy(ht_hbm.at[pl.ds(fset * NF, NF)], tab_v)

    fvecs = [jnp.full((16,), f, jnp.int32) for f in range(NF)]

    def round_body(r, carry):
        pltpu.sync_copy(src_hbm.at[g, pl.ds(r * RCHUNKS, RCHUNKS)], src_v)
        pltpu.sync_copy(dst_hbm.at[g, pl.ds(r * RCHUNKS, RCHUNKS)], dst_v)

        def body(j, carry2):
            for kk in range(CL // 16):
                sv = src_v[j, pl.ds(kk * 16, 16)]
                dv = dst_v[j, pl.ds(kk * 16, 16)]
                for f in range(NF):
                    vals = plsc.load_gather(tab_v, [fvecs[f], sv])
                    plsc.addupdate_scatter(agg_v, [fvecs[f], dv], vals)
            return carry2

        lax.fori_loop(0, RCHUNKS, body, 0)
        return carry

    lax.fori_loop(0, ROUNDS, round_body, 0)

    pltpu.sync_copy(agg_v, aggp_hbm.at[g, pl.ds(fset * NF, NF)])


def _aggregate(ht, src_g, dst_g, zeros):
    mesh = plsc.VectorSubcoreMesh(core_axis_name="c", subcore_axis_name="s")
    return pl.kernel(
        _agg_body,
        out_type=jax.ShapeDtypeStruct((NG, D_H, NPAD), jnp.float32),
        mesh=mesh,
        compiler_params=pltpu.CompilerParams(use_tc_tiling_on_sc=False, needs_layout_passes=False),
        scratch_types=[
            pltpu.VMEM((NF, NPAD), jnp.float32),
            pltpu.VMEM((NF, NPAD), jnp.float32),
            pltpu.VMEM((RCHUNKS, CL), jnp.int32),
            pltpu.VMEM((RCHUNKS, CL), jnp.int32),
        ],
    )(ht, src_g, dst_g, zeros)


def _hraw_kernel(wt_ref, ft_ref, h_ref):
    h_ref[...] = jnp.dot(wt_ref[...], ft_ref[...],
                         preferred_element_type=jnp.float32)


def _h_raw_t(WT, features_t):
    bn = 2048
    grid = NPAD // bn
    return pl.pallas_call(
        _hraw_kernel,
        grid=(grid,),
        in_specs=[
            pl.BlockSpec((D_H, D_IN), lambda i: (0, 0)),
            pl.BlockSpec((D_IN, bn), lambda i: (0, i)),
        ],
        out_specs=pl.BlockSpec((D_H, bn), lambda i: (0, i)),
        out_shape=jax.ShapeDtypeStruct((D_H, NPAD), jnp.float32),
    )(WT, features_t)


def _scale_kernel(h_ref, degs_ref, o_ref):
    deg = jnp.sum(degs_ref[0], axis=0)
    norm = lax.rsqrt(jnp.clip(deg, 1.0, None))
    o_ref[...] = h_ref[...] * norm[None, :]


def _h_scale_t(ht_raw, degs):
    bn = 2048
    grid = NPAD // bn
    return pl.pallas_call(
        _scale_kernel,
        grid=(grid,),
        in_specs=[
            pl.BlockSpec((D_H, bn), lambda i: (0, i)),
            pl.BlockSpec((2, NW, bn), lambda i: (0, 0, i)),
        ],
        out_specs=pl.BlockSpec((D_H, bn), lambda i: (0, i)),
        out_shape=jax.ShapeDtypeStruct((D_H, NPAD), jnp.float32),
    )(ht_raw, degs)


def _xt_kernel(aggp_ref, degs_ref, b_ref, xt_ref):
    aggt = jnp.sum(aggp_ref[...], axis=0)
    deg = jnp.sum(degs_ref[1], axis=0)
    norm = lax.rsqrt(jnp.clip(deg, 1.0, None))
    xt = aggt * norm[None, :] + b_ref[...][:, None]
    xt_ref[...] = xt[:, :N]


def _make_xt(aggp, degs, b):
    return pl.pallas_call(
        _xt_kernel,
        out_shape=jax.ShapeDtypeStruct((D_H, N), jnp.float32),
    )(aggp, degs, b)


def _adj_kernel(x_ref, xt_ref, o_ref):
    o_ref[...] = jnp.dot(x_ref[...], xt_ref[...],
                         preferred_element_type=jnp.float32)


def _decode(x, xt):
    bm = 2048
    bn = 2048
    gi = (N + bm - 1) // bm
    gj = (N + bn - 1) // bn
    return pl.pallas_call(
        _adj_kernel,
        grid=(gi, gj),
        in_specs=[
            pl.BlockSpec((bm, D_H), lambda i, j: (i, 0)),
            pl.BlockSpec((D_H, bn), lambda i, j: (0, j)),
        ],
        out_specs=pl.BlockSpec((bm, bn), lambda i, j: (i, j)),
        out_shape=jax.ShapeDtypeStruct((N, N), jnp.float32),
    )(x, xt)


@jax.jit
def kernel(features, edge_index, W, b):
    src = edge_index[0]
    dst = edge_index[1]
    pad = jnp.full((EP - E,), N, dtype=jnp.int32)
    src_flat = jnp.concatenate([src, pad])
    dst_flat = jnp.concatenate([dst, pad])
    src_w = src_flat.reshape(NW, CHUNKS, CL)
    dst_w = dst_flat.reshape(NW, CHUNKS, CL)
    src_g = src_flat.reshape(NG, GCHUNKS, CL)
    dst_g = dst_flat.reshape(NG, GCHUNKS, CL)
    features_t = jnp.concatenate(
        [features, jnp.zeros((NPAD - N, D_IN), jnp.float32)], axis=0).T

    zeros = jnp.zeros((NF, NPAD), jnp.float32)

    ht_raw = _h_raw_t(W.T, features_t)
    degs = _degrees(src_w, dst_w, zeros)
    ht = _h_scale_t(ht_raw, degs)
    aggp = _aggregate(ht, src_g, dst_g, zeros)
    xt = _make_xt(aggp, degs, b)
    x = xt.T
    adj = _decode(x, xt)
    return (adj, x)


# speedup vs baseline: 1.1695x; 1.1695x over previous
import jax
import jax.numpy as jnp
from jax import lax
from jax.experimental import pallas as pl
from jax.experimental.pallas import tpu as pltpu
from jax.experimental.pallas import tpu_sc as plsc

N = 10000
E = 320000
D_IN = 128
D_H = 16

NC = 2
NS = 16
NW = NC * NS
NPAD = 10240

CL = 128
CHUNKS = 80
EWP = CHUNKS * CL
EP = NW * EWP

NG = 8
GCHUNKS = EP // NG // CL
ROUNDS = 4
RCHUNKS = GCHUNKS // ROUNDS
NF = D_H // (NW // NG)

def _deg_body(src_hbm, dst_hbm, zeros_hbm, degs_hbm, src_v, dst_v, hs_v, hd_v):
    c = lax.axis_index("c")
    s = lax.axis_index("s")
    wid = c * NS + s

    zero = jnp.zeros((16,), jnp.float32)

    @plsc.parallel_loop(0, NPAD // 16, unroll=4)
    def zbody(i):
        hs_v[pl.ds(i * 16, 16)] = zero
        hd_v[pl.ds(i * 16, 16)] = zero

    pltpu.sync_copy(src_hbm.at[wid], src_v)
    pltpu.sync_copy(dst_hbm.at[wid], dst_v)

    ones = jnp.ones((16,), jnp.float32)

    @plsc.parallel_loop(0, CHUNKS, unroll=2)
    def body(j):
        for kk in range(CL // 16):
            sv = src_v[j, pl.ds(kk * 16, 16)]
            dv = dst_v[j, pl.ds(kk * 16, 16)]
            plsc.addupdate_scatter(hs_v, [sv], ones)
            plsc.addupdate_scatter(hd_v, [dv], ones)

    pltpu.sync_copy(hs_v, degs_hbm.at[0, wid])
    pltpu.sync_copy(hd_v, degs_hbm.at[1, wid])


def _degrees(src_p, dst_p, zeros):
    mesh = plsc.VectorSubcoreMesh(core_axis_name="c", subcore_axis_name="s")
    return pl.kernel(
        _deg_body,
        out_type=jax.ShapeDtypeStruct((2, NW, NPAD), jnp.float32),
        mesh=mesh,
        compiler_params=pltpu.CompilerParams(use_tc_tiling_on_sc=False, needs_layout_passes=False),
        scratch_types=[
            pltpu.VMEM((CHUNKS, CL), jnp.int32),
            pltpu.VMEM((CHUNKS, CL), jnp.int32),
            pltpu.VMEM((NPAD,), jnp.float32),
            pltpu.VMEM((NPAD,), jnp.float32),
        ],
    )(src_p, dst_p, zeros)


def _agg_body(ht_hbm, src_hbm, dst_hbm, zeros_hbm, aggp_hbm, tab_v, agg_v,
              src_v, dst_v):
    c = lax.axis_index("c")
    s = lax.axis_index("s")
    wid = c * NS + s
    g = lax.rem(wid, NG)
    fset = lax.div(wid, NG)

    zero = jnp.zeros((16,), jnp.float32)

    @plsc.parallel_loop(0, NPAD // 16, unroll=4)
    def zbody(i):
        for f in range(NF):
            agg_v[f, pl.ds(i * 16, 16)] = zero

    pltpu.sync_copy(ht_hbm.at[pl.ds(fset * NF, NF)], tab_v)

    fvecs = [jnp.full((16,), f, jnp.int32) for f in range(NF)]

    def round_body(r, carry):
        pltpu.sync_copy(src_hbm.at[g, pl.ds(r * RCHUNKS, RCHUNKS)], src_v)
        pltpu.sync_copy(dst_hbm.at[g, pl.ds(r * RCHUNKS, RCHUNKS)], dst_v)

        @plsc.parallel_loop(0, RCHUNKS, unroll=2)
        def body(j):
            for kk in range(CL // 16):
                sv = src_v[j, pl.ds(kk * 16, 16)]
                dv = dst_v[j, pl.ds(kk * 16, 16)]
                for f in range(NF):
                    vals = plsc.load_gather(tab_v, [fvecs[f], sv])
                    plsc.addupdate_scatter(agg_v, [fvecs[f], dv], vals)
        return carry

    lax.fori_loop(0, ROUNDS, round_body, 0)

    pltpu.sync_copy(agg_v, aggp_hbm.at[g, pl.ds(fset * NF, NF)])


def _aggregate(ht, src_g, dst_g, zeros):
    mesh = plsc.VectorSubcoreMesh(core_axis_name="c", subcore_axis_name="s")
    return pl.kernel(
        _agg_body,
        out_type=jax.ShapeDtypeStruct((NG, D_H, NPAD), jnp.float32),
        mesh=mesh,
        compiler_params=pltpu.CompilerParams(use_tc_tiling_on_sc=False, needs_layout_passes=False),
        scratch_types=[
            pltpu.VMEM((NF, NPAD), jnp.float32),
            pltpu.VMEM((NF, NPAD), jnp.float32),
            pltpu.VMEM((RCHUNKS, CL), jnp.int32),
            pltpu.VMEM((RCHUNKS, CL), jnp.int32),
        ],
    )(ht, src_g, dst_g, zeros)


def _hraw_kernel(wt_ref, ft_ref, h_ref):
    h_ref[...] = jnp.dot(wt_ref[...], ft_ref[...],
                         preferred_element_type=jnp.float32)


def _h_raw_t(WT, features_t):
    bn = 2048
    grid = NPAD // bn
    return pl.pallas_call(
        _hraw_kernel,
        grid=(grid,),
        in_specs=[
            pl.BlockSpec((D_H, D_IN), lambda i: (0, 0)),
            pl.BlockSpec((D_IN, bn), lambda i: (0, i)),
        ],
        out_specs=pl.BlockSpec((D_H, bn), lambda i: (0, i)),
        out_shape=jax.ShapeDtypeStruct((D_H, NPAD), jnp.float32),
    )(WT, features_t)


def _scale_kernel(h_ref, degs_ref, o_ref):
    deg = jnp.sum(degs_ref[0], axis=0)
    norm = lax.rsqrt(jnp.clip(deg, 1.0, None))
    o_ref[...] = h_ref[...] * norm[None, :]


def _h_scale_t(ht_raw, degs):
    bn = 2048
    grid = NPAD // bn
    return pl.pallas_call(
        _scale_kernel,
        grid=(grid,),
        in_specs=[
            pl.BlockSpec((D_H, bn), lambda i: (0, i)),
            pl.BlockSpec((2, NW, bn), lambda i: (0, 0, i)),
        ],
        out_specs=pl.BlockSpec((D_H, bn), lambda i: (0, i)),
        out_shape=jax.ShapeDtypeStruct((D_H, NPAD), jnp.float32),
    )(ht_raw, degs)


def _xt_kernel(aggp_ref, degs_ref, b_ref, xt_ref):
    aggt = jnp.sum(aggp_ref[...], axis=0)
    deg = jnp.sum(degs_ref[1], axis=0)
    norm = lax.rsqrt(jnp.clip(deg, 1.0, None))
    xt = aggt * norm[None, :] + b_ref[...][:, None]
    xt_ref[...] = xt[:, :N]


def _make_xt(aggp, degs, b):
    return pl.pallas_call(
        _xt_kernel,
        out_shape=jax.ShapeDtypeStruct((D_H, N), jnp.float32),
    )(aggp, degs, b)


def _adj_kernel(x_ref, xt_ref, o_ref):
    o_ref[...] = jnp.dot(x_ref[...], xt_ref[...],
                         preferred_element_type=jnp.float32)


def _decode(x, xt):
    bm = 2048
    bn = 2048
    gi = (N + bm - 1) // bm
    gj = (N + bn - 1) // bn
    return pl.pallas_call(
        _adj_kernel,
        grid=(gi, gj),
        in_specs=[
            pl.BlockSpec((bm, D_H), lambda i, j: (i, 0)),
            pl.BlockSpec((D_H, bn), lambda i, j: (0, j)),
        ],
        out_specs=pl.BlockSpec((bm, bn), lambda i, j: (i, j)),
        out_shape=jax.ShapeDtypeStruct((N, N), jnp.float32),
    )(x, xt)


@jax.jit
def kernel(features, edge_index, W, b):
    src = edge_index[0]
    dst = edge_index[1]
    pad = jnp.full((EP - E,), N, dtype=jnp.int32)
    src_flat = jnp.concatenate([src, pad])
    dst_flat = jnp.concatenate([dst, pad])
    src_w = src_flat.reshape(NW, CHUNKS, CL)
    dst_w = dst_flat.reshape(NW, CHUNKS, CL)
    src_g = src_flat.reshape(NG, GCHUNKS, CL)
    dst_g = dst_flat.reshape(NG, GCHUNKS, CL)
    features_t = jnp.concatenate(
        [features, jnp.zeros((NPAD - N, D_IN), jnp.float32)], axis=0).T

    zeros = jnp.zeros((NF, NPAD), jnp.float32)

    ht_raw = _h_raw_t(W.T, features_t)
    degs = _degrees(src_w, dst_w, zeros)
    ht = _h_scale_t(ht_raw, degs)
    aggp = _aggregate(ht, src_g, dst_g, zeros)
    xt = _make_xt(aggp, degs, b)
    x = xt.T
    adj = _decode(x, xt)
    return (adj, x)
